# Initial kernel scaffold; baseline (speedup 1.0000x reference)
import jax
import jax.numpy as jnp
from jax.experimental import pallas as pl


def kernel(x, edge_index, batch, W1a, b1a, W1b, b1b, W2a, b2a, W2b, b2b, Wl1, bl1, Wl2, bl2):
    raise NotImplementedError("write your pallas kernel here")


# speedup vs baseline: 3.7166x; 3.7166x over previous
import functools

import jax
import jax.numpy as jnp
from jax import lax
from jax.experimental import pallas as pl
from jax.experimental.pallas import tpu as pltpu
from jax.experimental.pallas import tpu_sc as plsc

NC, NS, LANES = 2, 16, 16
N = 10000
E = 160000
C = 256
H = C // 2
NG = 64

EPT = E // NS
K = 80
NCHUNK = EPT // K
RPT = N // NS

BR = 1000
GRID = N // BR


def _agg_body(table, src, dst, out, src_v, dst_v, rows_v, acc, sem):
    c = lax.axis_index("c")
    s = lax.axis_index("s")
    roff = s * RPT
    pltpu.sync_copy(table.at[pl.ds(c * N + roff, RPT)], acc.at[pl.ds(roff, RPT)])
    plsc.subcore_barrier()

    e0 = s * EPT
    coff = c * N

    def chunk(i, carry):
        base = e0 + i * K
        pltpu.sync_copy(src.at[pl.ds(base, K)], src_v)
        pltpu.sync_copy(dst.at[pl.ds(base, K)], dst_v)
        for j in range(K // LANES):
            sl = pl.ds(j * LANES, LANES)
            src_v[sl] = src_v[sl] + coff
        pltpu.async_copy(table.at[src_v], rows_v, sem).wait()
        pltpu.sync_copy(rows_v, acc.at[dst_v], add=True)
        return carry

    lax.fori_loop(0, NCHUNK, chunk, 0)
    plsc.subcore_barrier()
    pltpu.sync_copy(acc.at[pl.ds(roff, RPT)], out.at[pl.ds(c * N + roff, RPT)])


@functools.cache
def _make_agg():
    return pl.kernel(
        _agg_body,
        out_type=jax.ShapeDtypeStruct((2 * N, H), jnp.float32),
        mesh=plsc.VectorSubcoreMesh(
            core_axis_name="c", subcore_axis_name="s", num_cores=NC, num_subcores=NS
        ),
        scratch_types=[
            pltpu.VMEM((K,), jnp.int32),
            pltpu.VMEM((K,), jnp.int32),
            pltpu.VMEM((K, H), jnp.float32),
            pltpu.VMEM_SHARED((N, H), jnp.float32),
            pltpu.SemaphoreType.DMA,
        ],
        compiler_params=pltpu.CompilerParams(use_tc_tiling_on_sc=False),
    )


def _agg(table, src, dst):
    return _make_agg()(table, src, dst)


def _mlp1_body(s_ref, wa_ref, ba_ref, wb_ref, bb_ref, o_ref):
    blk = jnp.concatenate([s_ref[0], s_ref[1]], axis=1)
    t = jnp.maximum(
        jnp.dot(blk, wa_ref[...], preferred_element_type=jnp.float32) + ba_ref[...], 0.0
    )
    u = jnp.maximum(
        jnp.dot(t, wb_ref[...], preferred_element_type=jnp.float32) + bb_ref[...], 0.0
    )
    o_ref[0] = u[:, :H]
    o_ref[1] = u[:, H:]


def _mlp1(s_split, wa, ba, wb, bb):
    return pl.pallas_call(
        _mlp1_body,
        grid=(GRID,),
        in_specs=[
            pl.BlockSpec((2, BR, H), lambda i: (0, i, 0)),
            pl.BlockSpec((C, C), lambda i: (0, 0)),
            pl.BlockSpec((1, C), lambda i: (0, 0)),
            pl.BlockSpec((C, C), lambda i: (0, 0)),
            pl.BlockSpec((1, C), lambda i: (0, 0)),
        ],
        out_specs=pl.BlockSpec((2, BR, H), lambda i: (0, i, 0)),
        out_shape=jax.ShapeDtypeStruct((2, N, H), jnp.float32),
    )(s_split, wa, ba, wb, bb)


def _mlp2_body(s_ref, b_ref, wa_ref, ba_ref, wb_ref, bb_ref, o_ref):
    i = pl.program_id(0)
    blk = jnp.concatenate([s_ref[0], s_ref[1]], axis=1)
    t = jnp.maximum(
        jnp.dot(blk, wa_ref[...], preferred_element_type=jnp.float32) + ba_ref[...], 0.0
    )
    u = jnp.maximum(
        jnp.dot(t, wb_ref[...], preferred_element_type=jnp.float32) + bb_ref[...], 0.0
    )
    bvec = b_ref[0, 0]
    onehot = (
        bvec[:, None] == lax.broadcasted_iota(jnp.int32, (BR, NG), 1)
    ).astype(jnp.float32)
    dims = (((0,), (0,)), ((), ()))
    part = lax.dot_general(onehot, u, dims, preferred_element_type=jnp.float32)
    cpart = lax.dot_general(
        onehot, jnp.ones_like(u), dims, preferred_element_type=jnp.float32
    )
    upd = jnp.concatenate([part, cpart], axis=0)

    @pl.when(i == 0)
    def _():
        o_ref[...] = jnp.zeros_like(o_ref)

    o_ref[...] += upd


def _mlp2(s_split, batch3, wa, ba, wb, bb):
    return pl.pallas_call(
        _mlp2_body,
        grid=(GRID,),
        in_specs=[
            pl.BlockSpec((2, BR, H), lambda i: (0, i, 0)),
            pl.BlockSpec((1, 1, BR), lambda i: (i, 0, 0)),
            pl.BlockSpec((C, C), lambda i: (0, 0)),
            pl.BlockSpec((1, C), lambda i: (0, 0)),
            pl.BlockSpec((C, C), lambda i: (0, 0)),
            pl.BlockSpec((1, C), lambda i: (0, 0)),
        ],
        out_specs=pl.BlockSpec((2 * NG, C), lambda i: (0, 0)),
        out_shape=jax.ShapeDtypeStruct((2 * NG, C), jnp.float32),
    )(s_split, batch3, wa, ba, wb, bb)


def _head_body(gc_ref, wl1_ref, bl1_ref, wl2_ref, bl2_ref, o_ref):
    sums = gc_ref[:NG, :]
    cnts = gc_ref[NG:, :]
    g = sums / jnp.maximum(cnts, 1.0)
    z = jnp.maximum(
        jnp.dot(g, wl1_ref[...], preferred_element_type=jnp.float32) + bl1_ref[...], 0.0
    )
    r = jnp.sum(z * wl2_ref[...], axis=1, keepdims=True) + bl2_ref[...]
    o_ref[...] = jnp.broadcast_to(r, (NG, H))


def _head(gc, wl1, bl1, wl2r, bl2r):
    return pl.pallas_call(
        _head_body,
        grid=(1,),
        in_specs=[
            pl.BlockSpec((2 * NG, C), lambda i: (0, 0)),
            pl.BlockSpec((C, H), lambda i: (0, 0)),
            pl.BlockSpec((1, H), lambda i: (0, 0)),
            pl.BlockSpec((1, H), lambda i: (0, 0)),
            pl.BlockSpec((1, H), lambda i: (0, 0)),
        ],
        out_specs=pl.BlockSpec((NG, H), lambda i: (0, 0)),
        out_shape=jax.ShapeDtypeStruct((NG, H), jnp.float32),
    )(gc, wl1, bl1, wl2r, bl2r)


def kernel(x, edge_index, batch, W1a, b1a, W1b, b1b, W2a, b2a, W2b, b2b, Wl1, bl1, Wl2, bl2):
    src = edge_index[0]
    dst = edge_index[1]
    x_split = jnp.concatenate([x[:, :H], x[:, H:]], axis=0)

    s1 = _agg(x_split, src, dst)
    h1 = _mlp1(s1.reshape(2, N, H), W1a, b1a.reshape(1, C), W1b, b1b.reshape(1, C))
    s2 = _agg(h1.reshape(2 * N, H), src, dst)
    gc = _mlp2(
        s2.reshape(2, N, H),
        batch.reshape(GRID, 1, BR),
        W2a, b2a.reshape(1, C), W2b, b2b.reshape(1, C),
    )
    res = _head(
        gc,
        Wl1, bl1.reshape(1, H),
        Wl2.reshape(1, H),
        jnp.broadcast_to(bl2.reshape(1, 1), (1, H)),
    )
    return res[:, :1]


# speedup vs baseline: 8.4268x; 2.2673x over previous
import functools

import jax
import jax.numpy as jnp
from jax import lax
from jax.experimental import pallas as pl
from jax.experimental.pallas import tpu as pltpu
from jax.experimental.pallas import tpu_sc as plsc

NC, NS, LANES = 2, 16, 16
N = 10000
E = 160000
C = 256
H = C // 2
NG = 64

EPT = E // NS
K = 100
NCHUNK = EPT // K
PAIRS = NCHUNK // 2
EROWS = E // K
RPT = N // NS

BR = 1000
GRID = N // BR


def _agg_body(table, srcs, dst, out, src_big, dst_big, rows_a, rows_b, acc, sem_a, sem_b):
    c = lax.axis_index("c")
    s = lax.axis_index("s")
    roff = s * RPT
    pltpu.sync_copy(table.at[pl.ds(c * N + roff, RPT)], acc.at[pl.ds(roff, RPT)])
    pltpu.sync_copy(srcs.at[pl.ds(c * EROWS + s * NCHUNK, NCHUNK)], src_big)
    pltpu.sync_copy(dst.at[pl.ds(s * NCHUNK, NCHUNK)], dst_big)

    def start(i, buf, sem):
        pltpu.make_async_copy(table.at[src_big.at[i]], buf, sem).start()

    def finish(buf, sem, i):
        pltpu.make_async_copy(table.at[src_big.at[0]], buf, sem).wait()
        pltpu.sync_copy(buf, acc.at[dst_big.at[i]], add=True)

    start(0, rows_a, sem_a)
    plsc.subcore_barrier()

    def pair(p, carry):
        i0 = p * 2
        start(i0 + 1, rows_b, sem_b)
        finish(rows_a, sem_a, i0)

        @pl.when(p < PAIRS - 1)
        def _():
            start(i0 + 2, rows_a, sem_a)

        finish(rows_b, sem_b, i0 + 1)
        return carry

    lax.fori_loop(0, PAIRS, pair, 0)
    plsc.subcore_barrier()
    pltpu.sync_copy(acc.at[pl.ds(roff, RPT)], out.at[pl.ds(c * N + roff, RPT)])


@functools.cache
def _make_agg():
    return pl.kernel(
        _agg_body,
        out_type=jax.ShapeDtypeStruct((2 * N, H), jnp.float32),
        mesh=plsc.VectorSubcoreMesh(
            core_axis_name="c", subcore_axis_name="s", num_cores=NC, num_subcores=NS
        ),
        scratch_types=[
            pltpu.VMEM((NCHUNK, K), jnp.int32),
            pltpu.VMEM((NCHUNK, K), jnp.int32),
            pltpu.VMEM((K, H), jnp.float32),
            pltpu.VMEM((K, H), jnp.float32),
            pltpu.VMEM_SHARED((N, H), jnp.float32),
            pltpu.SemaphoreType.DMA,
            pltpu.SemaphoreType.DMA,
        ],
        compiler_params=pltpu.CompilerParams(use_tc_tiling_on_sc=False),
    )


def _agg(table, src, dst):
    return _make_agg()(table, src, dst)


def _mlp1_body(s_ref, wa_ref, ba_ref, wb_ref, bb_ref, o_ref):
    blk = jnp.concatenate([s_ref[0], s_ref[1]], axis=1)
    t = jnp.maximum(
        jnp.dot(blk, wa_ref[...], preferred_element_type=jnp.float32) + ba_ref[...], 0.0
    )
    u = jnp.maximum(
        jnp.dot(t, wb_ref[...], preferred_element_type=jnp.float32) + bb_ref[...], 0.0
    )
    o_ref[0] = u[:, :H]
    o_ref[1] = u[:, H:]


def _mlp1(s_split, wa, ba, wb, bb):
    return pl.pallas_call(
        _mlp1_body,
        grid=(GRID,),
        in_specs=[
            pl.BlockSpec((2, BR, H), lambda i: (0, i, 0)),
            pl.BlockSpec((C, C), lambda i: (0, 0)),
            pl.BlockSpec((1, C), lambda i: (0, 0)),
            pl.BlockSpec((C, C), lambda i: (0, 0)),
            pl.BlockSpec((1, C), lambda i: (0, 0)),
        ],
        out_specs=pl.BlockSpec((2, BR, H), lambda i: (0, i, 0)),
        out_shape=jax.ShapeDtypeStruct((2, N, H), jnp.float32),
    )(s_split, wa, ba, wb, bb)


def _mlp2_body(s_ref, b_ref, wa_ref, ba_ref, wb_ref, bb_ref, o_ref):
    i = pl.program_id(0)
    blk = jnp.concatenate([s_ref[0], s_ref[1]], axis=1)
    t = jnp.maximum(
        jnp.dot(blk, wa_ref[...], preferred_element_type=jnp.float32) + ba_ref[...], 0.0
    )
    u = jnp.maximum(
        jnp.dot(t, wb_ref[...], preferred_element_type=jnp.float32) + bb_ref[...], 0.0
    )
    bvec = b_ref[0, 0]
    onehot = (
        bvec[:, None] == lax.broadcasted_iota(jnp.int32, (BR, NG), 1)
    ).astype(jnp.float32)
    dims = (((0,), (0,)), ((), ()))
    part = lax.dot_general(onehot, u, dims, preferred_element_type=jnp.float32)
    cpart = lax.dot_general(
        onehot, jnp.ones_like(u), dims, preferred_element_type=jnp.float32
    )
    upd = jnp.concatenate([part, cpart], axis=0)

    @pl.when(i == 0)
    def _():
        o_ref[...] = jnp.zeros_like(o_ref)

    o_ref[...] += upd


def _mlp2(s_split, batch3, wa, ba, wb, bb):
    return pl.pallas_call(
        _mlp2_body,
        grid=(GRID,),
        in_specs=[
            pl.BlockSpec((2, BR, H), lambda i: (0, i, 0)),
            pl.BlockSpec((1, 1, BR), lambda i: (i, 0, 0)),
            pl.BlockSpec((C, C), lambda i: (0, 0)),
            pl.BlockSpec((1, C), lambda i: (0, 0)),
            pl.BlockSpec((C, C), lambda i: (0, 0)),
            pl.BlockSpec((1, C), lambda i: (0, 0)),
        ],
        out_specs=pl.BlockSpec((2 * NG, C), lambda i: (0, 0)),
        out_shape=jax.ShapeDtypeStruct((2 * NG, C), jnp.float32),
    )(s_split, batch3, wa, ba, wb, bb)


def _head_body(gc_ref, wl1_ref, bl1_ref, wl2_ref, bl2_ref, o_ref):
    sums = gc_ref[:NG, :]
    cnts = gc_ref[NG:, :]
    g = sums / jnp.maximum(cnts, 1.0)
    z = jnp.maximum(
        jnp.dot(g, wl1_ref[...], preferred_element_type=jnp.float32) + bl1_ref[...], 0.0
    )
    r = jnp.sum(z * wl2_ref[...], axis=1, keepdims=True) + bl2_ref[...]
    o_ref[...] = jnp.broadcast_to(r, (NG, H))


def _head(gc, wl1, bl1, wl2r, bl2r):
    return pl.pallas_call(
        _head_body,
        grid=(1,),
        in_specs=[
            pl.BlockSpec((2 * NG, C), lambda i: (0, 0)),
            pl.BlockSpec((C, H), lambda i: (0, 0)),
            pl.BlockSpec((1, H), lambda i: (0, 0)),
            pl.BlockSpec((1, H), lambda i: (0, 0)),
            pl.BlockSpec((1, H), lambda i: (0, 0)),
        ],
        out_specs=pl.BlockSpec((NG, H), lambda i: (0, 0)),
        out_shape=jax.ShapeDtypeStruct((NG, H), jnp.float32),
    )(gc, wl1, bl1, wl2r, bl2r)


def kernel(x, edge_index, batch, W1a, b1a, W1b, b1b, W2a, b2a, W2b, b2b, Wl1, bl1, Wl2, bl2):
    src = edge_index[0]
    dst = edge_index[1]
    x_split = jnp.concatenate([x[:, :H], x[:, H:]], axis=0)
    srcs = jnp.concatenate([src, src + N]).reshape(2 * EROWS, K)
    dst2 = dst.reshape(EROWS, K)

    s1 = _agg(x_split, srcs, dst2)
    h1 = _mlp1(s1.reshape(2, N, H), W1a, b1a.reshape(1, C), W1b, b1b.reshape(1, C))
    s2 = _agg(h1.reshape(2 * N, H), srcs, dst2)
    gc = _mlp2(
        s2.reshape(2, N, H),
        batch.reshape(GRID, 1, BR),
        W2a, b2a.reshape(1, C), W2b, b2b.reshape(1, C),
    )
    res = _head(
        gc,
        Wl1, bl1.reshape(1, H),
        Wl2.reshape(1, H),
        jnp.broadcast_to(bl2.reshape(1, 1), (1, H)),
    )
    return res[:, :1]
